# baseline (device time: 122513 ns/iter reference)
import jax
import jax.numpy as jnp
from jax import lax
from jax.experimental import pallas as pl
from jax.experimental.pallas import tpu as pltpu

N_DEV = 4
SQ = 1024
SKV_LOC = 1024
HQ = 8
DH = 128
D = HQ * DH
SCALE = 0.08838834764831843
BLK = 64


def kernel(x, Wq, K_ext, V_ext, Wo):
    x2 = x.reshape(SQ, D)
    k2 = K_ext.reshape(SKV_LOC, D)
    v2 = V_ext.reshape(SKV_LOC, D)

    def body(x_ref, wq_ref, k_ref, v_ref, wo_ref, out_ref,
             acc_ref, lacc_ref, ctx_comm, l_comm,
             ctx_send_sems, ctx_recv_sems, l_send_sems, l_recv_sems):
        my = lax.axis_index("i")
        left = (my - 1) % N_DEV
        right = (my + 1) % N_DEV

        barrier_sem = pltpu.get_barrier_semaphore()
        for nbr in (left, right):
            pl.semaphore_signal(
                barrier_sem, inc=1,
                device_id=(nbr,), device_id_type=pl.DeviceIdType.MESH,
            )
        pl.semaphore_wait(barrier_sem, 2)

        xb = x_ref[...].astype(jnp.bfloat16)
        wqb = wq_ref[...].astype(jnp.bfloat16)
        q = jnp.dot(xb, wqb, preferred_element_type=jnp.float32)
        qb = (q * SCALE).astype(jnp.bfloat16)

        rows = lax.broadcasted_iota(jnp.int32, (SQ, SKV_LOC), 0) // BLK
        cols = (lax.broadcasted_iota(jnp.int32, (SQ, SKV_LOC), 1)
                + my * SKV_LOC) // BLK
        mask = (rows == cols) | (cols == 0) | (((rows + cols) % 3) == 0)

        l_cols = []
        for h in range(HQ):
            sl = slice(h * DH, (h + 1) * DH)
            k_h = k_ref[:, sl].astype(jnp.bfloat16)
            v_h = v_ref[:, sl].astype(jnp.bfloat16)
            s = lax.dot_general(
                qb[:, sl], k_h, (((1,), (1,)), ((), ())),
                preferred_element_type=jnp.float32,
            )
            w = jnp.exp(jnp.where(mask, s, -1e9))
            l_cols.append(jnp.sum(w, axis=1, keepdims=True))
            ctx_h = jnp.dot(w.astype(jnp.bfloat16), v_h,
                            preferred_element_type=jnp.float32)
            acc_ref[:, sl] = ctx_h
            ctx_comm[0, :, sl] = ctx_h.astype(jnp.bfloat16)
        lloc = jnp.concatenate(l_cols, axis=1)
        lacc_ref[...] = lloc
        l_comm[0, :, :] = lloc

        for h in range(N_DEV - 1):
            s_slot = h % 2
            r_slot = (h + 1) % 2
            rdma_ctx = pltpu.make_async_remote_copy(
                src_ref=ctx_comm.at[s_slot],
                dst_ref=ctx_comm.at[r_slot],
                send_sem=ctx_send_sems.at[s_slot],
                recv_sem=ctx_recv_sems.at[r_slot],
                device_id=(right,), device_id_type=pl.DeviceIdType.MESH,
            )
            rdma_l = pltpu.make_async_remote_copy(
                src_ref=l_comm.at[s_slot],
                dst_ref=l_comm.at[r_slot],
                send_sem=l_send_sems.at[s_slot],
                recv_sem=l_recv_sems.at[r_slot],
                device_id=(right,), device_id_type=pl.DeviceIdType.MESH,
            )
            rdma_ctx.start()
            rdma_l.start()
            rdma_ctx.wait()
            rdma_l.wait()
            acc_ref[...] = acc_ref[...] + ctx_comm[r_slot].astype(jnp.float32)
            lacc_ref[...] = lacc_ref[...] + l_comm[r_slot]

        norm_cols = []
        for h in range(HQ):
            sl = slice(h * DH, (h + 1) * DH)
            norm_cols.append(acc_ref[:, sl] / lacc_ref[:, h:h + 1])
        normed = jnp.concatenate(norm_cols, axis=1).astype(jnp.bfloat16)
        wob = wo_ref[...].astype(jnp.bfloat16)
        out_ref[...] = jnp.dot(normed, wob, preferred_element_type=jnp.float32)

    out = pl.pallas_call(
        body,
        out_shape=jax.ShapeDtypeStruct((SQ, D), jnp.float32),
        in_specs=[pl.BlockSpec(memory_space=pltpu.VMEM)] * 5,
        out_specs=pl.BlockSpec(memory_space=pltpu.VMEM),
        scratch_shapes=[
            pltpu.VMEM((SQ, D), jnp.float32),
            pltpu.VMEM((SQ, HQ), jnp.float32),
            pltpu.VMEM((2, SQ, D), jnp.bfloat16),
            pltpu.VMEM((2, SQ, HQ), jnp.float32),
            pltpu.SemaphoreType.DMA((2,)),
            pltpu.SemaphoreType.DMA((2,)),
            pltpu.SemaphoreType.DMA((2,)),
            pltpu.SemaphoreType.DMA((2,)),
        ],
        compiler_params=pltpu.CompilerParams(collective_id=0),
    )(x2, Wq, k2, v2, Wo)
    return out.reshape(1, SQ, D)


# device time: 64666 ns/iter; 1.8946x vs baseline; 1.8946x over previous
import jax
import jax.numpy as jnp
from jax import lax
from jax.experimental import pallas as pl
from jax.experimental.pallas import tpu as pltpu

N_DEV = 4
SQ = 1024
SKV_LOC = 1024
HQ = 8
DH = 128
D = HQ * DH
HALF = D // 2
SCALE = 0.08838834764831843
BLK = 64


def kernel(x, Wq, K_ext, V_ext, Wo):
    x2 = x.reshape(SQ, D)
    k2 = K_ext.reshape(SKV_LOC, D)
    v2 = V_ext.reshape(SKV_LOC, D)

    def body(x_ref, wq_ref, k_ref, v_ref, wo_ref, out_ref,
             a_self, a_in1, a_sum, a_in2,
             b_self, b_in1, b_sum, b_in2,
             l_self, l_in1, l_sum, l_in2,
             send_sems, recv_sems):
        my = lax.axis_index("i")
        p1 = my ^ 1
        p2 = 3 - my

        barrier_sem = pltpu.get_barrier_semaphore()
        for nbr in (p1, p2):
            pl.semaphore_signal(
                barrier_sem, inc=1,
                device_id=(nbr,), device_id_type=pl.DeviceIdType.MESH,
            )
        pl.semaphore_wait(barrier_sem, 2)

        xb = x_ref[...].astype(jnp.bfloat16)
        wqb = wq_ref[...].astype(jnp.bfloat16)
        q = jnp.dot(xb, wqb, preferred_element_type=jnp.float32)
        qb = (q * SCALE).astype(jnp.bfloat16)

        rows = lax.broadcasted_iota(jnp.int32, (SQ, SKV_LOC), 0) // BLK
        cols = (lax.broadcasted_iota(jnp.int32, (SQ, SKV_LOC), 1)
                + my * SKV_LOC) // BLK
        mask = (rows == cols) | (cols == 0) | (((rows + cols) % 3) == 0)

        def head_partial(h, dst, dst_off):
            sl = slice(h * DH, (h + 1) * DH)
            k_h = k_ref[:, sl].astype(jnp.bfloat16)
            v_h = v_ref[:, sl].astype(jnp.bfloat16)
            s = lax.dot_general(
                qb[:, sl], k_h, (((1,), (1,)), ((), ())),
                preferred_element_type=jnp.float32,
            )
            w = jnp.exp(jnp.where(mask, s, -1e9))
            ctx_h = jnp.dot(w.astype(jnp.bfloat16), v_h,
                            preferred_element_type=jnp.float32)
            dst[:, h * DH - dst_off:(h + 1) * DH - dst_off] = (
                ctx_h.astype(jnp.bfloat16))
            return jnp.sum(w, axis=1, keepdims=True)

        l_cols = []
        for h in range(4):
            l_cols.append(head_partial(h, a_self, 0))

        rd_a1 = pltpu.make_async_remote_copy(
            src_ref=a_self, dst_ref=a_in1,
            send_sem=send_sems.at[0], recv_sem=recv_sems.at[0],
            device_id=(p1,), device_id_type=pl.DeviceIdType.MESH,
        )
        rd_a1.start()

        for h in range(4, 8):
            l_cols.append(head_partial(h, b_self, HALF))
        l_self[...] = jnp.concatenate(l_cols, axis=1)

        rd_b1 = pltpu.make_async_remote_copy(
            src_ref=b_self, dst_ref=b_in1,
            send_sem=send_sems.at[1], recv_sem=recv_sems.at[1],
            device_id=(p2,), device_id_type=pl.DeviceIdType.MESH,
        )
        rd_b1.start()
        rd_l1 = pltpu.make_async_remote_copy(
            src_ref=l_self, dst_ref=l_in1,
            send_sem=send_sems.at[4], recv_sem=recv_sems.at[4],
            device_id=(p1,), device_id_type=pl.DeviceIdType.MESH,
        )
        rd_l1.start()

        rd_a1.wait_recv()
        a_sum[...] = (a_self[...].astype(jnp.float32)
                      + a_in1[...].astype(jnp.float32)).astype(jnp.bfloat16)
        rd_a2 = pltpu.make_async_remote_copy(
            src_ref=a_sum, dst_ref=a_in2,
            send_sem=send_sems.at[2], recv_sem=recv_sems.at[2],
            device_id=(p2,), device_id_type=pl.DeviceIdType.MESH,
        )
        rd_a2.start()

        rd_l1.wait_recv()
        l_sum[...] = l_self[...] + l_in1[...]
        rd_l2 = pltpu.make_async_remote_copy(
            src_ref=l_sum, dst_ref=l_in2,
            send_sem=send_sems.at[5], recv_sem=recv_sems.at[5],
            device_id=(p2,), device_id_type=pl.DeviceIdType.MESH,
        )
        rd_l2.start()

        rd_b1.wait_recv()
        b_sum[...] = (b_self[...].astype(jnp.float32)
                      + b_in1[...].astype(jnp.float32)).astype(jnp.bfloat16)
        rd_b2 = pltpu.make_async_remote_copy(
            src_ref=b_sum, dst_ref=b_in2,
            send_sem=send_sems.at[3], recv_sem=recv_sems.at[3],
            device_id=(p1,), device_id_type=pl.DeviceIdType.MESH,
        )
        rd_b2.start()

        wob = wo_ref[...].astype(jnp.bfloat16)
        rd_a2.wait_recv()
        rd_l2.wait_recv()
        l_tot = l_sum[...] + l_in2[...]
        na_cols = []
        for h in range(4):
            sl = slice(h * DH, (h + 1) * DH)
            a_tot = (a_sum[:, sl].astype(jnp.float32)
                     + a_in2[:, sl].astype(jnp.float32))
            na_cols.append(a_tot / l_tot[:, h:h + 1])
        norm_a = jnp.concatenate(na_cols, axis=1).astype(jnp.bfloat16)
        out_a = jnp.dot(norm_a, wob[:HALF, :],
                        preferred_element_type=jnp.float32)

        rd_b2.wait_recv()
        nb_cols = []
        for h in range(4, 8):
            sl = slice(h * DH - HALF, (h + 1) * DH - HALF)
            b_tot = (b_sum[:, sl].astype(jnp.float32)
                     + b_in2[:, sl].astype(jnp.float32))
            nb_cols.append(b_tot / l_tot[:, h:h + 1])
        norm_b = jnp.concatenate(nb_cols, axis=1).astype(jnp.bfloat16)
        out_ref[...] = out_a + jnp.dot(norm_b, wob[HALF:, :],
                                       preferred_element_type=jnp.float32)

        for rd in (rd_a1, rd_b1, rd_l1, rd_a2, rd_l2, rd_b2):
            rd.wait_send()

    out = pl.pallas_call(
        body,
        out_shape=jax.ShapeDtypeStruct((SQ, D), jnp.float32),
        in_specs=[pl.BlockSpec(memory_space=pltpu.VMEM)] * 5,
        out_specs=pl.BlockSpec(memory_space=pltpu.VMEM),
        scratch_shapes=[
            pltpu.VMEM((SQ, HALF), jnp.bfloat16),
            pltpu.VMEM((SQ, HALF), jnp.bfloat16),
            pltpu.VMEM((SQ, HALF), jnp.bfloat16),
            pltpu.VMEM((SQ, HALF), jnp.bfloat16),
            pltpu.VMEM((SQ, HALF), jnp.bfloat16),
            pltpu.VMEM((SQ, HALF), jnp.bfloat16),
            pltpu.VMEM((SQ, HALF), jnp.bfloat16),
            pltpu.VMEM((SQ, HALF), jnp.bfloat16),
            pltpu.VMEM((SQ, HQ), jnp.float32),
            pltpu.VMEM((SQ, HQ), jnp.float32),
            pltpu.VMEM((SQ, HQ), jnp.float32),
            pltpu.VMEM((SQ, HQ), jnp.float32),
            pltpu.SemaphoreType.DMA((6,)),
            pltpu.SemaphoreType.DMA((6,)),
        ],
        compiler_params=pltpu.CompilerParams(collective_id=0),
    )(x2, Wq, k2, v2, Wo)
    return out.reshape(1, SQ, D)


# device time: 62094 ns/iter; 1.9730x vs baseline; 1.0414x over previous
import jax
import jax.numpy as jnp
from jax import lax
from jax.experimental import pallas as pl
from jax.experimental.pallas import tpu as pltpu

N_DEV = 4
SQ = 1024
SKV_LOC = 1024
HQ = 8
DH = 128
D = HQ * DH
NC = 4
CW = D // NC
SCALE = 0.08838834764831843
BLK = 64


def kernel(x, Wq, K_ext, V_ext, Wo):
    x2 = x.reshape(SQ, D)
    k2 = K_ext.reshape(SKV_LOC, D)
    v2 = V_ext.reshape(SKV_LOC, D)

    def body(x_ref, wq_ref, k_ref, v_ref, wo_ref, out_ref,
             cself, cin1, csum, cin2,
             l_self, l_in1, l_sum, l_in2,
             s1_send, s1_recv, s2_send, s2_recv, l_send, l_recv):
        my = lax.axis_index("i")
        p1 = my ^ 1
        p2 = 3 - my

        barrier_sem = pltpu.get_barrier_semaphore()
        for nbr in (p1, p2):
            pl.semaphore_signal(
                barrier_sem, inc=1,
                device_id=(nbr,), device_id_type=pl.DeviceIdType.MESH,
            )
        pl.semaphore_wait(barrier_sem, 2)

        xb = x_ref[...].astype(jnp.bfloat16)
        wqb = wq_ref[...].astype(jnp.bfloat16)
        q = jnp.dot(xb, wqb, preferred_element_type=jnp.float32)
        qb = (q * SCALE).astype(jnp.bfloat16)

        rows = lax.broadcasted_iota(jnp.int32, (SQ, SKV_LOC), 0) // BLK
        cols = (lax.broadcasted_iota(jnp.int32, (SQ, SKV_LOC), 1)
                + my * SKV_LOC) // BLK
        mask = (rows == cols) | (cols == 0) | (((rows + cols) % 3) == 0)

        def head_partial(h, c):
            sl = slice(h * DH, (h + 1) * DH)
            k_h = k_ref[:, sl].astype(jnp.bfloat16)
            v_h = v_ref[:, sl].astype(jnp.bfloat16)
            s = lax.dot_general(
                qb[:, sl], k_h, (((1,), (1,)), ((), ())),
                preferred_element_type=jnp.float32,
            )
            w = jnp.exp(jnp.where(mask, s, -1e9))
            ctx_h = jnp.dot(w.astype(jnp.bfloat16), v_h,
                            preferred_element_type=jnp.float32)
            off = (h % 2) * DH
            cself[c, :, off:off + DH] = ctx_h.astype(jnp.bfloat16)
            return jnp.sum(w, axis=1, keepdims=True)

        def partner_of(c, stage):
            return p1 if (c % 2 == 0) == (stage == 1) else p2

        l_cols = []
        rd_s1 = []
        for c in range(NC):
            l_cols.append(head_partial(2 * c, c))
            l_cols.append(head_partial(2 * c + 1, c))
            rd = pltpu.make_async_remote_copy(
                src_ref=cself.at[c], dst_ref=cin1.at[c],
                send_sem=s1_send.at[c], recv_sem=s1_recv.at[c],
                device_id=(partner_of(c, 1),),
                device_id_type=pl.DeviceIdType.MESH,
            )
            rd.start()
            rd_s1.append(rd)

        l_self[...] = jnp.concatenate(l_cols, axis=1)
        rd_l1 = pltpu.make_async_remote_copy(
            src_ref=l_self, dst_ref=l_in1,
            send_sem=l_send.at[0], recv_sem=l_recv.at[0],
            device_id=(p1,), device_id_type=pl.DeviceIdType.MESH,
        )
        rd_l1.start()

        rd_s2 = []
        for c in range(NC):
            rd_s1[c].wait_recv()
            csum[c] = (cself[c].astype(jnp.float32)
                       + cin1[c].astype(jnp.float32)).astype(jnp.bfloat16)
            rd = pltpu.make_async_remote_copy(
                src_ref=csum.at[c], dst_ref=cin2.at[c],
                send_sem=s2_send.at[c], recv_sem=s2_recv.at[c],
                device_id=(partner_of(c, 2),),
                device_id_type=pl.DeviceIdType.MESH,
            )
            rd.start()
            rd_s2.append(rd)

        rd_l1.wait_recv()
        l_sum[...] = l_self[...] + l_in1[...]
        rd_l2 = pltpu.make_async_remote_copy(
            src_ref=l_sum, dst_ref=l_in2,
            send_sem=l_send.at[1], recv_sem=l_recv.at[1],
            device_id=(p2,), device_id_type=pl.DeviceIdType.MESH,
        )
        rd_l2.start()
        rd_l2.wait_recv()
        l_tot = l_sum[...] + l_in2[...]

        wob = wo_ref[...].astype(jnp.bfloat16)
        acc = None
        for c in range(NC):
            rd_s2[c].wait_recv()
            tot = (csum[c].astype(jnp.float32)
                   + cin2[c].astype(jnp.float32))
            n_cols = []
            for j in range(2):
                h = 2 * c + j
                n_cols.append(tot[:, j * DH:(j + 1) * DH]
                              / l_tot[:, h:h + 1])
            norm_c = jnp.concatenate(n_cols, axis=1).astype(jnp.bfloat16)
            part = jnp.dot(norm_c, wob[c * CW:(c + 1) * CW, :],
                           preferred_element_type=jnp.float32)
            acc = part if acc is None else acc + part
        out_ref[...] = acc

        for rd in rd_s1 + rd_s2 + [rd_l1, rd_l2]:
            rd.wait_send()

    out = pl.pallas_call(
        body,
        out_shape=jax.ShapeDtypeStruct((SQ, D), jnp.float32),
        in_specs=[pl.BlockSpec(memory_space=pltpu.VMEM)] * 5,
        out_specs=pl.BlockSpec(memory_space=pltpu.VMEM),
        scratch_shapes=[
            pltpu.VMEM((NC, SQ, CW), jnp.bfloat16),
            pltpu.VMEM((NC, SQ, CW), jnp.bfloat16),
            pltpu.VMEM((NC, SQ, CW), jnp.bfloat16),
            pltpu.VMEM((NC, SQ, CW), jnp.bfloat16),
            pltpu.VMEM((SQ, HQ), jnp.float32),
            pltpu.VMEM((SQ, HQ), jnp.float32),
            pltpu.VMEM((SQ, HQ), jnp.float32),
            pltpu.VMEM((SQ, HQ), jnp.float32),
            pltpu.SemaphoreType.DMA((NC,)),
            pltpu.SemaphoreType.DMA((NC,)),
            pltpu.SemaphoreType.DMA((NC,)),
            pltpu.SemaphoreType.DMA((NC,)),
            pltpu.SemaphoreType.DMA((2,)),
            pltpu.SemaphoreType.DMA((2,)),
        ],
        compiler_params=pltpu.CompilerParams(collective_id=0),
    )(x2, Wq, k2, v2, Wo)
    return out.reshape(1, SQ, D)


# device time: 59118 ns/iter; 2.0723x vs baseline; 1.0503x over previous
import jax
import jax.numpy as jnp
from jax import lax
from jax.experimental import pallas as pl
from jax.experimental.pallas import tpu as pltpu

N_DEV = 4
SQ = 1024
SKV_LOC = 1024
HQ = 8
DH = 128
D = HQ * DH
NC = 4
CW = D // NC
SCALE = 0.08838834764831843
BLK = 64


def kernel(x, Wq, K_ext, V_ext, Wo):
    x2 = x.reshape(SQ, D)
    k2 = K_ext.reshape(SKV_LOC, D)
    v2 = V_ext.reshape(SKV_LOC, D)

    def body(x_ref, wq_ref, k_ref, v_ref, wo_ref, out_ref,
             cself, cin1, csum, cin2,
             l_self, l_in1, l_sum, l_in2,
             s1_send, s1_recv, s2_send, s2_recv, l_send, l_recv):
        my = lax.axis_index("i")
        p1 = my ^ 1
        p2 = 3 - my

        barrier_sem = pltpu.get_barrier_semaphore()
        for nbr in (p1, p2):
            pl.semaphore_signal(
                barrier_sem, inc=1,
                device_id=(nbr,), device_id_type=pl.DeviceIdType.MESH,
            )
        pl.semaphore_wait(barrier_sem, 2)

        xb = x_ref[...].astype(jnp.bfloat16)
        wqb = wq_ref[...].astype(jnp.bfloat16)
        q = jnp.dot(xb, wqb, preferred_element_type=jnp.float32)
        qb = (q * SCALE).astype(jnp.bfloat16)

        rows = lax.broadcasted_iota(jnp.int32, (SQ, SKV_LOC), 0) // BLK
        cols = (lax.broadcasted_iota(jnp.int32, (SQ, SKV_LOC), 1)
                + my * SKV_LOC) // BLK
        mask = (rows == cols) | (cols == 0) | (((rows + cols) % 3) == 0)

        def head_partial(h, c):
            sl = slice(h * DH, (h + 1) * DH)
            k_h = k_ref[:, sl].astype(jnp.bfloat16)
            v_h = v_ref[:, sl].astype(jnp.bfloat16)
            s = lax.dot_general(
                qb[:, sl], k_h, (((1,), (1,)), ((), ())),
                preferred_element_type=jnp.float32,
            )
            w = jnp.exp(jnp.where(mask, s, -1e9))
            ctx_h = jnp.dot(w.astype(jnp.bfloat16), v_h,
                            preferred_element_type=jnp.float32)
            off = (h % 2) * DH
            cself[c, :, off:off + DH] = ctx_h.astype(jnp.bfloat16)
            return jnp.sum(w, axis=1, keepdims=True)

        def partner_of(c, stage):
            return p1 if (c % 2 == 0) == (stage == 1) else p2

        l_cols = []
        rd_s1 = []
        for c in range(NC):
            l_cols.append(head_partial(2 * c, c))
            l_cols.append(head_partial(2 * c + 1, c))
            rd = pltpu.make_async_remote_copy(
                src_ref=cself.at[c], dst_ref=cin1.at[c],
                send_sem=s1_send.at[c], recv_sem=s1_recv.at[c],
                device_id=(partner_of(c, 1),),
                device_id_type=pl.DeviceIdType.MESH,
            )
            rd.start()
            rd_s1.append(rd)

        l_cols.append(jnp.zeros((SQ, 128 - HQ), jnp.float32))
        l_self[...] = jnp.concatenate(l_cols, axis=1).astype(jnp.bfloat16)
        rd_l1 = pltpu.make_async_remote_copy(
            src_ref=l_self, dst_ref=l_in1,
            send_sem=l_send.at[0], recv_sem=l_recv.at[0],
            device_id=(p1,), device_id_type=pl.DeviceIdType.MESH,
        )
        rd_l1.start()

        rd_s2 = []
        for c in range(NC):
            rd_s1[c].wait_recv()
            csum[c] = (cself[c].astype(jnp.float32)
                       + cin1[c].astype(jnp.float32)).astype(jnp.bfloat16)
            rd = pltpu.make_async_remote_copy(
                src_ref=csum.at[c], dst_ref=cin2.at[c],
                send_sem=s2_send.at[c], recv_sem=s2_recv.at[c],
                device_id=(partner_of(c, 2),),
                device_id_type=pl.DeviceIdType.MESH,
            )
            rd.start()
            rd_s2.append(rd)

        rd_l1.wait_recv()
        l_sum[...] = (l_self[...].astype(jnp.float32)
                      + l_in1[...].astype(jnp.float32)).astype(jnp.bfloat16)
        rd_l2 = pltpu.make_async_remote_copy(
            src_ref=l_sum, dst_ref=l_in2,
            send_sem=l_send.at[1], recv_sem=l_recv.at[1],
            device_id=(p2,), device_id_type=pl.DeviceIdType.MESH,
        )
        rd_l2.start()
        rd_l2.wait_recv()
        l_tot = (l_sum[...].astype(jnp.float32)
                 + l_in2[...].astype(jnp.float32))

        wob = wo_ref[...].astype(jnp.bfloat16)
        acc = None
        for c in range(NC):
            rd_s2[c].wait_recv()
            tot = (csum[c].astype(jnp.float32)
                   + cin2[c].astype(jnp.float32))
            n_cols = []
            for j in range(2):
                h = 2 * c + j
                n_cols.append(tot[:, j * DH:(j + 1) * DH]
                              / l_tot[:, h:h + 1])
            norm_c = jnp.concatenate(n_cols, axis=1).astype(jnp.bfloat16)
            part = jnp.dot(norm_c, wob[c * CW:(c + 1) * CW, :],
                           preferred_element_type=jnp.float32)
            acc = part if acc is None else acc + part
        out_ref[...] = acc

        for rd in rd_s1 + rd_s2 + [rd_l1, rd_l2]:
            rd.wait_send()

    out = pl.pallas_call(
        body,
        out_shape=jax.ShapeDtypeStruct((SQ, D), jnp.float32),
        in_specs=[pl.BlockSpec(memory_space=pltpu.VMEM)] * 5,
        out_specs=pl.BlockSpec(memory_space=pltpu.VMEM),
        scratch_shapes=[
            pltpu.VMEM((NC, SQ, CW), jnp.bfloat16),
            pltpu.VMEM((NC, SQ, CW), jnp.bfloat16),
            pltpu.VMEM((NC, SQ, CW), jnp.bfloat16),
            pltpu.VMEM((NC, SQ, CW), jnp.bfloat16),
            pltpu.VMEM((SQ, 128), jnp.bfloat16),
            pltpu.VMEM((SQ, 128), jnp.bfloat16),
            pltpu.VMEM((SQ, 128), jnp.bfloat16),
            pltpu.VMEM((SQ, 128), jnp.bfloat16),
            pltpu.SemaphoreType.DMA((NC,)),
            pltpu.SemaphoreType.DMA((NC,)),
            pltpu.SemaphoreType.DMA((NC,)),
            pltpu.SemaphoreType.DMA((NC,)),
            pltpu.SemaphoreType.DMA((2,)),
            pltpu.SemaphoreType.DMA((2,)),
        ],
        compiler_params=pltpu.CompilerParams(collective_id=0),
    )(x2, Wq, k2, v2, Wo)
    return out.reshape(1, SQ, D)


# device time: 58127 ns/iter; 2.1077x vs baseline; 1.0170x over previous
import jax
import jax.numpy as jnp
from jax import lax
from jax.experimental import pallas as pl
from jax.experimental.pallas import tpu as pltpu

N_DEV = 4
SQ = 1024
SKV_LOC = 1024
HQ = 8
DH = 128
D = HQ * DH
NC = 4
CW = D // NC
SCALE = 0.08838834764831843
BLK = 64


def kernel(x, Wq, K_ext, V_ext, Wo):
    k2 = K_ext.astype(jnp.bfloat16).reshape(SKV_LOC, D)
    v2 = V_ext.astype(jnp.bfloat16).reshape(SKV_LOC, D)

    def body(x_ref, wq_ref, k_ref, v_ref, wo_ref, out_ref,
             cself, cin1, csum, cin2,
             l_self, l_in1, l_sum, l_in2,
             s1_send, s1_recv, s2_send, s2_recv, l_send, l_recv):
        my = lax.axis_index("i")
        p1 = my ^ 1
        p2 = 3 - my

        barrier_sem = pltpu.get_barrier_semaphore()
        for nbr in (p1, p2):
            pl.semaphore_signal(
                barrier_sem, inc=1,
                device_id=(nbr,), device_id_type=pl.DeviceIdType.MESH,
            )

        xb = x_ref[0].astype(jnp.bfloat16)
        wqb = wq_ref[...].astype(jnp.bfloat16)
        q = jnp.dot(xb, wqb, preferred_element_type=jnp.float32)
        qb = (q * SCALE).astype(jnp.bfloat16)

        rows = lax.broadcasted_iota(jnp.int32, (SQ, SKV_LOC), 0) // BLK
        cols = (lax.broadcasted_iota(jnp.int32, (SQ, SKV_LOC), 1)
                + my * SKV_LOC) // BLK
        mask = (rows == cols) | (cols == 0) | (((rows + cols) % 3) == 0)

        def head_partial(h, c):
            sl = slice(h * DH, (h + 1) * DH)
            s = lax.dot_general(
                qb[:, sl], k_ref[:, sl], (((1,), (1,)), ((), ())),
                preferred_element_type=jnp.float32,
            )
            w = jnp.exp(jnp.where(mask, s, -1e9))
            ctx_h = jnp.dot(w.astype(jnp.bfloat16), v_ref[:, sl],
                            preferred_element_type=jnp.float32)
            off = (h % 2) * DH
            cself[c, :, off:off + DH] = ctx_h.astype(jnp.bfloat16)
            return jnp.sum(w, axis=1, keepdims=True)

        def partner_of(c, stage):
            return p1 if (c % 2 == 0) == (stage == 1) else p2

        def start_s1(c):
            rd = pltpu.make_async_remote_copy(
                src_ref=cself.at[c], dst_ref=cin1.at[c],
                send_sem=s1_send.at[c], recv_sem=s1_recv.at[c],
                device_id=(partner_of(c, 1),),
                device_id_type=pl.DeviceIdType.MESH,
            )
            rd.start()
            return rd

        def pairsum_start_s2(c, rd1):
            rd1.wait_recv()
            csum[c] = (cself[c].astype(jnp.float32)
                       + cin1[c].astype(jnp.float32)).astype(jnp.bfloat16)
            rd = pltpu.make_async_remote_copy(
                src_ref=csum.at[c], dst_ref=cin2.at[c],
                send_sem=s2_send.at[c], recv_sem=s2_recv.at[c],
                device_id=(partner_of(c, 2),),
                device_id_type=pl.DeviceIdType.MESH,
            )
            rd.start()
            return rd

        l_cols = []
        l_cols.append(head_partial(0, 0))
        l_cols.append(head_partial(1, 0))
        pl.semaphore_wait(barrier_sem, 2)
        rd1_0 = start_s1(0)

        l_cols.append(head_partial(2, 1))
        l_cols.append(head_partial(3, 1))
        rd1_1 = start_s1(1)
        l_cols.append(head_partial(4, 2))
        l_cols.append(head_partial(5, 2))
        rd1_2 = start_s1(2)
        rd2_0 = pairsum_start_s2(0, rd1_0)

        l_cols.append(head_partial(6, 3))
        l_cols.append(head_partial(7, 3))
        rd1_3 = start_s1(3)
        rd2_1 = pairsum_start_s2(1, rd1_1)

        l_cols.append(jnp.zeros((SQ, 128 - HQ), jnp.float32))
        l_self[...] = jnp.concatenate(l_cols, axis=1).astype(jnp.bfloat16)
        rd_l1 = pltpu.make_async_remote_copy(
            src_ref=l_self, dst_ref=l_in1,
            send_sem=l_send.at[0], recv_sem=l_recv.at[0],
            device_id=(p1,), device_id_type=pl.DeviceIdType.MESH,
        )
        rd_l1.start()

        rd2_2 = pairsum_start_s2(2, rd1_2)
        rd2_3 = pairsum_start_s2(3, rd1_3)

        rd_l1.wait_recv()
        l_sum[...] = (l_self[...].astype(jnp.float32)
                      + l_in1[...].astype(jnp.float32)).astype(jnp.bfloat16)
        rd_l2 = pltpu.make_async_remote_copy(
            src_ref=l_sum, dst_ref=l_in2,
            send_sem=l_send.at[1], recv_sem=l_recv.at[1],
            device_id=(p2,), device_id_type=pl.DeviceIdType.MESH,
        )
        rd_l2.start()
        rd_l2.wait_recv()
        l_tot = (l_sum[...].astype(jnp.float32)
                 + l_in2[...].astype(jnp.float32))

        wob = wo_ref[...].astype(jnp.bfloat16)
        acc = None
        for c, rd2 in enumerate((rd2_0, rd2_1, rd2_2, rd2_3)):
            rd2.wait_recv()
            tot = (csum[c].astype(jnp.float32)
                   + cin2[c].astype(jnp.float32))
            n_cols = []
            for j in range(2):
                h = 2 * c + j
                n_cols.append(tot[:, j * DH:(j + 1) * DH]
                              / l_tot[:, h:h + 1])
            norm_c = jnp.concatenate(n_cols, axis=1).astype(jnp.bfloat16)
            part = jnp.dot(norm_c, wob[c * CW:(c + 1) * CW, :],
                           preferred_element_type=jnp.float32)
            acc = part if acc is None else acc + part
        out_ref[0] = acc

        for rd in (rd1_0, rd1_1, rd1_2, rd1_3,
                   rd2_0, rd2_1, rd2_2, rd2_3, rd_l1, rd_l2):
            rd.wait_send()

    out = pl.pallas_call(
        body,
        out_shape=jax.ShapeDtypeStruct((1, SQ, D), jnp.float32),
        in_specs=[pl.BlockSpec(memory_space=pltpu.VMEM)] * 5,
        out_specs=pl.BlockSpec(memory_space=pltpu.VMEM),
        scratch_shapes=[
            pltpu.VMEM((NC, SQ, CW), jnp.bfloat16),
            pltpu.VMEM((NC, SQ, CW), jnp.bfloat16),
            pltpu.VMEM((NC, SQ, CW), jnp.bfloat16),
            pltpu.VMEM((NC, SQ, CW), jnp.bfloat16),
            pltpu.VMEM((SQ, 128), jnp.bfloat16),
            pltpu.VMEM((SQ, 128), jnp.bfloat16),
            pltpu.VMEM((SQ, 128), jnp.bfloat16),
            pltpu.VMEM((SQ, 128), jnp.bfloat16),
            pltpu.SemaphoreType.DMA((NC,)),
            pltpu.SemaphoreType.DMA((NC,)),
            pltpu.SemaphoreType.DMA((NC,)),
            pltpu.SemaphoreType.DMA((NC,)),
            pltpu.SemaphoreType.DMA((2,)),
            pltpu.SemaphoreType.DMA((2,)),
        ],
        compiler_params=pltpu.CompilerParams(collective_id=0),
    )(x, Wq, k2, v2, Wo)
    return out
